# baseline (speedup 1.0000x reference)
import functools
import jax
import jax.numpy as jnp
from jax import lax
from jax.experimental import pallas as pl
from jax.experimental.pallas import tpu as pltpu
from jax.experimental.pallas import tpu_sc as plsc

_SCALE = 0.95
_THRESH = 0.01

_N = 4096
_TOT = _N * _N
_NW = 32
_PER_W = _TOT // _NW
_CHUNK = 32768
_NCHUNK = _PER_W // _CHUNK
_NBUF = 3
_GROUPS = _CHUNK // 64


def _compute_chunk(wbuf, mbuf, cnt, cj):
    def gbody(g, acc):
        words = mbuf[pl.ds(g * 16, 16)]
        base = g * 64
        for j in range(4):
            idx = base + cj[j]
            wv = plsc.load_gather(wbuf, [idx])
            sv = wv * _SCALE
            weak = jnp.abs(sv) < _THRESH
            imp0 = (words & (1 << (8 * j))) == 0
            prune = weak & imp0
            plsc.store_scatter(wbuf, [idx], jnp.where(prune, 0.0, sv))
            acc = acc + plsc.all_reduce_population_count(prune)
        return acc

    return lax.fori_loop(0, _GROUPS, gbody, cnt)


def _sc_body(w_hbm, m_hbm, out_hbm, part_hbm,
             w0, w1, w2, m0, m1, m2, cnt_buf,
             si0, si1, si2, so0, so1, so2):
    cid = lax.axis_index("c")
    sid = lax.axis_index("s")
    wid = sid * 2 + cid
    base = pl.multiple_of(wid * _PER_W, _CHUNK)

    wbufs = (w0, w1, w2)
    mbufs = (m0, m1, m2)
    sis = (si0, si1, si2)
    sos = (so0, so1, so2)

    iota = lax.iota(jnp.int32, 16)
    cj = tuple(4 * iota + j for j in range(4))

    def start_in(t):
        b = t % _NBUF
        off = pl.multiple_of(base + t * _CHUNK, _CHUNK)
        moff = pl.multiple_of((base + t * _CHUNK) // 4, _CHUNK // 4)
        pltpu.async_copy(w_hbm.at[pl.ds(off, _CHUNK)], wbufs[b], sis[b])
        pltpu.async_copy(m_hbm.at[pl.ds(moff, _CHUNK // 4)], mbufs[b], sis[b])

    def wait_in(t):
        b = t % _NBUF
        off = pl.multiple_of(base + t * _CHUNK, _CHUNK)
        moff = pl.multiple_of((base + t * _CHUNK) // 4, _CHUNK // 4)
        pltpu.make_async_copy(w_hbm.at[pl.ds(off, _CHUNK)], wbufs[b], sis[b]).wait()
        pltpu.make_async_copy(m_hbm.at[pl.ds(moff, _CHUNK // 4)], mbufs[b], sis[b]).wait()

    def start_out(t):
        b = t % _NBUF
        off = pl.multiple_of(base + t * _CHUNK, _CHUNK)
        pltpu.async_copy(wbufs[b], out_hbm.at[pl.ds(off, _CHUNK)], sos[b])

    def wait_out(t):
        b = t % _NBUF
        off = pl.multiple_of(base + t * _CHUNK, _CHUNK)
        pltpu.make_async_copy(wbufs[b], out_hbm.at[pl.ds(off, _CHUNK)], sos[b]).wait()

    start_in(0)
    start_in(1)

    cnt = jnp.zeros((16,), jnp.int32)
    for t in range(_NCHUNK):
        wait_in(t)
        cnt = _compute_chunk(wbufs[t % _NBUF], mbufs[t % _NBUF], cnt, cj)
        start_out(t)
        if t + 2 < _NCHUNK:
            if t >= 1:
                wait_out(t - 1)
            start_in(t + 2)
    wait_out(_NCHUNK - 2)
    wait_out(_NCHUNK - 1)

    cnt_buf[...] = cnt
    pltpu.sync_copy(cnt_buf, part_hbm.at[wid])


@jax.jit
def kernel(weights, importance_mask):
    w_flat = weights.reshape(-1)
    m_words = importance_mask.view(jnp.int32).reshape(-1)

    mesh = plsc.VectorSubcoreMesh(core_axis_name="c", subcore_axis_name="s")
    out, part = pl.kernel(
        _sc_body,
        out_type=[
            jax.ShapeDtypeStruct((_TOT,), jnp.float32),
            jax.ShapeDtypeStruct((_NW, 16), jnp.int32),
        ],
        mesh=mesh,
        compiler_params=pltpu.CompilerParams(needs_layout_passes=False),
        scratch_types=[
            pltpu.VMEM((_CHUNK,), jnp.float32),
            pltpu.VMEM((_CHUNK,), jnp.float32),
            pltpu.VMEM((_CHUNK,), jnp.float32),
            pltpu.VMEM((_CHUNK // 4,), jnp.int32),
            pltpu.VMEM((_CHUNK // 4,), jnp.int32),
            pltpu.VMEM((_CHUNK // 4,), jnp.int32),
            pltpu.VMEM((16,), jnp.int32),
            pltpu.SemaphoreType.DMA,
            pltpu.SemaphoreType.DMA,
            pltpu.SemaphoreType.DMA,
            pltpu.SemaphoreType.DMA,
            pltpu.SemaphoreType.DMA,
            pltpu.SemaphoreType.DMA,
        ],
    )(w_flat, m_words)
    n_pruned = part[:, 0].sum().astype(jnp.int32)
    return out.reshape(_N, _N), n_pruned


# speedup vs baseline: 1.9173x; 1.9173x over previous
import jax
import jax.numpy as jnp
from jax import lax
from jax.experimental import pallas as pl
from jax.experimental.pallas import tpu as pltpu
from jax.experimental.pallas import tpu_sc as plsc

_SCALE = 0.95
_THRESH = 0.01

_N = 4096
_NW = 32
_ROWS_W = _N // _NW
_CH = 8
_NCHUNK = _ROWS_W // _CH
_NBUF = 3
_CHUNK = _CH * _N
_GROUPS = _CHUNK // 64
_MW = _N // 4


def _compute_chunk(wbuf, mbuf, cnt, cj):
    @plsc.parallel_loop(0, _GROUPS, unroll=4, carry=cnt)
    def gbody(g, acc):
        words = mbuf[pl.ds(g * 16, 16)]
        base = g * 64
        for j in range(4):
            idx = base + cj[j]
            wv = plsc.load_gather(wbuf, [idx])
            sv = wv * _SCALE
            weak = jnp.abs(sv) < _THRESH
            imp0 = (words & (1 << (8 * j))) == 0
            prune = weak & imp0
            plsc.store_scatter(wbuf, [idx], jnp.where(prune, 0.0, sv))
            acc = acc + plsc.all_reduce_population_count(prune)
        return acc

    return gbody


def _sc_body(w_hbm, m_hbm, out_hbm, part_hbm,
             w0, w1, w2, m0, m1, m2, cnt_buf,
             si0, si1, si2, so0, so1, so2):
    cid = lax.axis_index("c")
    sid = lax.axis_index("s")
    wid = sid * 2 + cid
    row0 = pl.multiple_of(wid * _ROWS_W, _ROWS_W)

    wbufs = (w0, w1, w2)
    mbufs = (m0, m1, m2)
    sis = (si0, si1, si2)
    sos = (so0, so1, so2)

    iota = lax.iota(jnp.int32, 16)
    cj = tuple(4 * iota + j for j in range(4))

    def start_in(t):
        b = t % _NBUF
        for r in range(_CH):
            row = row0 + t * _CH + r
            pltpu.async_copy(w_hbm.at[row], wbufs[b].at[pl.ds(r * _N, _N)], sis[b])
            pltpu.async_copy(m_hbm.at[row], mbufs[b].at[pl.ds(r * _MW, _MW)], sis[b])

    def wait_in(t):
        b = t % _NBUF
        for r in range(_CH):
            row = row0 + t * _CH + r
            pltpu.make_async_copy(w_hbm.at[row], wbufs[b].at[pl.ds(r * _N, _N)], sis[b]).wait()
            pltpu.make_async_copy(m_hbm.at[row], mbufs[b].at[pl.ds(r * _MW, _MW)], sis[b]).wait()

    def start_out(t):
        b = t % _NBUF
        for r in range(_CH):
            row = row0 + t * _CH + r
            pltpu.async_copy(wbufs[b].at[pl.ds(r * _N, _N)], out_hbm.at[row], sos[b])

    def wait_out(t):
        b = t % _NBUF
        for r in range(_CH):
            row = row0 + t * _CH + r
            pltpu.make_async_copy(wbufs[b].at[pl.ds(r * _N, _N)], out_hbm.at[row], sos[b]).wait()

    start_in(0)
    start_in(1)

    cnt = jnp.zeros((16,), jnp.int32)
    for t in range(_NCHUNK):
        wait_in(t)
        cnt = _compute_chunk(wbufs[t % _NBUF], mbufs[t % _NBUF], cnt, cj)
        start_out(t)
        if t + 2 < _NCHUNK:
            if t >= 1:
                wait_out(t - 1)
            start_in(t + 2)
    wait_out(_NCHUNK - 2)
    wait_out(_NCHUNK - 1)

    cnt_buf[...] = cnt
    pltpu.sync_copy(cnt_buf, part_hbm.at[wid])


@jax.jit
def kernel(weights, importance_mask):
    m_words = importance_mask.view(jnp.int32)

    mesh = plsc.VectorSubcoreMesh(core_axis_name="c", subcore_axis_name="s")
    out, part = pl.kernel(
        _sc_body,
        out_type=[
            jax.ShapeDtypeStruct((_N, _N), jnp.float32),
            jax.ShapeDtypeStruct((_NW, 16), jnp.int32),
        ],
        mesh=mesh,
        compiler_params=pltpu.CompilerParams(needs_layout_passes=False),
        scratch_types=[
            pltpu.VMEM((_CHUNK,), jnp.float32),
            pltpu.VMEM((_CHUNK,), jnp.float32),
            pltpu.VMEM((_CHUNK,), jnp.float32),
            pltpu.VMEM((_CHUNK // 4,), jnp.int32),
            pltpu.VMEM((_CHUNK // 4,), jnp.int32),
            pltpu.VMEM((_CHUNK // 4,), jnp.int32),
            pltpu.VMEM((16,), jnp.int32),
            pltpu.SemaphoreType.DMA,
            pltpu.SemaphoreType.DMA,
            pltpu.SemaphoreType.DMA,
            pltpu.SemaphoreType.DMA,
            pltpu.SemaphoreType.DMA,
            pltpu.SemaphoreType.DMA,
        ],
    )(weights, m_words)
    n_pruned = part[:, 0].sum().astype(jnp.int32)
    return out, n_pruned


# speedup vs baseline: 1.9187x; 1.0007x over previous
import jax
import jax.numpy as jnp
from jax import lax
from jax.experimental import pallas as pl
from jax.experimental.pallas import tpu as pltpu
from jax.experimental.pallas import tpu_sc as plsc

_SCALE = 0.95
_THRESH = 0.01

_N = 4096
_NW = 32
_ROWS_W = _N // _NW
_CH = 8
_NCHUNK = _ROWS_W // _CH
_NBUF = 3
_CHUNK = _CH * _N
_GROUPS = _CHUNK // 64
_MW = _N // 4


def _compute_chunk(wbuf, mbuf, cnt, cj):
    @plsc.parallel_loop(0, _GROUPS, unroll=4, carry=cnt)
    def gbody(g, acc):
        words = mbuf[pl.ds(g * 16, 16)]
        base = g * 64
        for j in range(4):
            idx = base + cj[j]
            wv = plsc.load_gather(wbuf, [idx])
            sv = wv * _SCALE
            weak = jnp.abs(sv) < _THRESH
            imp0 = (words & (1 << (8 * j))) == 0
            prune = weak & imp0
            plsc.store_scatter(wbuf, [idx], jnp.where(prune, 0.0, sv))
            acc = acc + plsc.all_reduce_population_count(prune)
        return acc

    return gbody


def _sc_body(w_hbm, m_hbm, out_hbm, part_hbm,
             w0, w1, w2, m0, m1, m2, cnt_buf,
             si0, si1, si2, so0, so1, so2):
    cid = lax.axis_index("c")
    sid = lax.axis_index("s")
    wid = sid * 2 + cid
    row0 = pl.multiple_of(wid * _ROWS_W, _ROWS_W)

    wbufs = (w0, w1, w2)
    mbufs = (m0, m1, m2)
    sis = (si0, si1, si2)
    sos = (so0, so1, so2)

    iota = lax.iota(jnp.int32, 16)
    cj = tuple(4 * iota + j for j in range(4))

    def start_in(t):
        b = t % _NBUF
        for r in range(_CH):
            row = row0 + t * _CH + r
            pltpu.async_copy(w_hbm.at[row], wbufs[b].at[pl.ds(r * _N, _N)], sis[b])
            pltpu.async_copy(m_hbm.at[row], mbufs[b].at[pl.ds(r * _MW, _MW)], sis[b])

    def wait_in(t):
        b = t % _NBUF
        for r in range(_CH):
            row = row0 + t * _CH + r
            pltpu.make_async_copy(w_hbm.at[row], wbufs[b].at[pl.ds(r * _N, _N)], sis[b]).wait()
            pltpu.make_async_copy(m_hbm.at[row], mbufs[b].at[pl.ds(r * _MW, _MW)], sis[b]).wait()

    def start_out(t):
        b = t % _NBUF
        for r in range(_CH):
            row = row0 + t * _CH + r
            pltpu.async_copy(wbufs[b].at[pl.ds(r * _N, _N)], out_hbm.at[row], sos[b])

    def wait_out(t):
        b = t % _NBUF
        for r in range(_CH):
            row = row0 + t * _CH + r
            pltpu.make_async_copy(wbufs[b].at[pl.ds(r * _N, _N)], out_hbm.at[row], sos[b]).wait()

    start_in(0)
    start_in(1)

    cnt = jnp.zeros((16,), jnp.int32)
    for t in range(_NCHUNK):
        wait_in(t)
        cnt = _compute_chunk(wbufs[t % _NBUF], mbufs[t % _NBUF], cnt, cj)
        start_out(t)
        if t + 2 < _NCHUNK:
            if t >= 1:
                wait_out(t - 1)
            start_in(t + 2)
    wait_out(_NCHUNK - 2)
    wait_out(_NCHUNK - 1)

    cnt_buf[...] = cnt
    pltpu.sync_copy(cnt_buf, part_hbm.at[wid])


@jax.jit
def kernel(weights, importance_mask):
    m_words = importance_mask.view(jnp.int32)

    mesh = plsc.VectorSubcoreMesh(core_axis_name="c", subcore_axis_name="s")
    out, part = pl.kernel(
        _sc_body,
        out_type=[
            jax.ShapeDtypeStruct((_N, _N), jnp.float32),
            jax.ShapeDtypeStruct((_NW, 16), jnp.int32),
        ],
        mesh=mesh,
        compiler_params=pltpu.CompilerParams(
            needs_layout_passes=False, use_tc_tiling_on_sc=True
        ),
        scratch_types=[
            pltpu.VMEM((_CHUNK,), jnp.float32),
            pltpu.VMEM((_CHUNK,), jnp.float32),
            pltpu.VMEM((_CHUNK,), jnp.float32),
            pltpu.VMEM((_CHUNK // 4,), jnp.int32),
            pltpu.VMEM((_CHUNK // 4,), jnp.int32),
            pltpu.VMEM((_CHUNK // 4,), jnp.int32),
            pltpu.VMEM((16,), jnp.int32),
            pltpu.SemaphoreType.DMA,
            pltpu.SemaphoreType.DMA,
            pltpu.SemaphoreType.DMA,
            pltpu.SemaphoreType.DMA,
            pltpu.SemaphoreType.DMA,
            pltpu.SemaphoreType.DMA,
        ],
    )(weights, m_words)
    n_pruned = part[:, 0].sum().astype(jnp.int32)
    return out, n_pruned


# speedup vs baseline: 4.5986x; 2.3968x over previous
import jax
import jax.numpy as jnp
from jax import lax
from jax.experimental import pallas as pl
from jax.experimental.pallas import tpu as pltpu
from jax.experimental.pallas import tpu_sc as plsc

_SCALE = 0.95
_THRESH = 0.01

_N = 4096
_NW = 32
_ROWS_W = _N // _NW
_CH = 8
_NCHUNK = _ROWS_W // _CH
_NBUF = 3
_CHUNK = _CH * _N
_GROUPS = _CHUNK // 64


def _compute_chunk(wbuf, mbuf, cnt, cj):
    @plsc.parallel_loop(0, _GROUPS, unroll=4, carry=cnt)
    def gbody(g, acc):
        words = plsc.bitcast(mbuf[pl.ds(g * 64, 64)], jnp.int32)
        base = g * 64
        for j in range(4):
            idx = base + cj[j]
            wv = plsc.load_gather(wbuf, [idx])
            sv = wv * _SCALE
            weak = jnp.abs(sv) < _THRESH
            imp0 = (words & (1 << (8 * j))) == 0
            prune = weak & imp0
            plsc.store_scatter(wbuf, [idx], jnp.where(prune, 0.0, sv))
            acc = acc + plsc.all_reduce_population_count(prune)
        return acc

    return gbody


def _sc_body(w_hbm, m_hbm, out_hbm, part_hbm,
             w0, w1, w2, m0, m1, m2, cnt_buf,
             si0, si1, si2, so0, so1, so2):
    cid = lax.axis_index("c")
    sid = lax.axis_index("s")
    wid = sid * 2 + cid
    row0 = wid * _ROWS_W

    wbufs = (w0, w1, w2)
    mbufs = (m0, m1, m2)
    sis = (si0, si1, si2)
    sos = (so0, so1, so2)

    iota = lax.iota(jnp.int32, 16)
    cj = tuple(4 * iota + j for j in range(4))

    def start_in(t):
        b = t % _NBUF
        for r in range(_CH):
            row = row0 + t * _CH + r
            pltpu.async_copy(w_hbm.at[row], wbufs[b].at[pl.ds(r * _N, _N)], sis[b])
        moff = pl.multiple_of((wid * (_ROWS_W // _CH) + t) * _CHUNK, _CHUNK)
        pltpu.async_copy(m_hbm.at[pl.ds(moff, _CHUNK)], mbufs[b], sis[b])

    def wait_in(t):
        b = t % _NBUF
        for r in range(_CH):
            row = row0 + t * _CH + r
            pltpu.make_async_copy(w_hbm.at[row], wbufs[b].at[pl.ds(r * _N, _N)], sis[b]).wait()
        moff = pl.multiple_of((wid * (_ROWS_W // _CH) + t) * _CHUNK, _CHUNK)
        pltpu.make_async_copy(m_hbm.at[pl.ds(moff, _CHUNK)], mbufs[b], sis[b]).wait()

    def start_out(t):
        b = t % _NBUF
        for r in range(_CH):
            row = row0 + t * _CH + r
            pltpu.async_copy(wbufs[b].at[pl.ds(r * _N, _N)], out_hbm.at[row], sos[b])

    def wait_out(t):
        b = t % _NBUF
        for r in range(_CH):
            row = row0 + t * _CH + r
            pltpu.make_async_copy(wbufs[b].at[pl.ds(r * _N, _N)], out_hbm.at[row], sos[b]).wait()

    start_in(0)
    start_in(1)

    cnt = jnp.zeros((16,), jnp.int32)
    for t in range(_NCHUNK):
        wait_in(t)
        cnt = _compute_chunk(wbufs[t % _NBUF], mbufs[t % _NBUF], cnt, cj)
        start_out(t)
        if t + 2 < _NCHUNK:
            if t >= 1:
                wait_out(t - 1)
            start_in(t + 2)
    wait_out(_NCHUNK - 2)
    wait_out(_NCHUNK - 1)

    cnt_buf[...] = cnt
    pltpu.sync_copy(cnt_buf, part_hbm.at[wid])


@jax.jit
def kernel(weights, importance_mask):
    m_bytes = importance_mask.view(jnp.int8).reshape(-1)

    mesh = plsc.VectorSubcoreMesh(core_axis_name="c", subcore_axis_name="s")
    out, part = pl.kernel(
        _sc_body,
        out_type=[
            jax.ShapeDtypeStruct((_N, _N), jnp.float32),
            jax.ShapeDtypeStruct((_NW, 16), jnp.int32),
        ],
        mesh=mesh,
        compiler_params=pltpu.CompilerParams(needs_layout_passes=False),
        scratch_types=[
            pltpu.VMEM((_CHUNK,), jnp.float32),
            pltpu.VMEM((_CHUNK,), jnp.float32),
            pltpu.VMEM((_CHUNK,), jnp.float32),
            pltpu.VMEM((_CHUNK,), jnp.int8),
            pltpu.VMEM((_CHUNK,), jnp.int8),
            pltpu.VMEM((_CHUNK,), jnp.int8),
            pltpu.VMEM((16,), jnp.int32),
            pltpu.SemaphoreType.DMA,
            pltpu.SemaphoreType.DMA,
            pltpu.SemaphoreType.DMA,
            pltpu.SemaphoreType.DMA,
            pltpu.SemaphoreType.DMA,
            pltpu.SemaphoreType.DMA,
        ],
    )(weights, m_bytes)
    n_pruned = part[:, 0].sum().astype(jnp.int32)
    return out, n_pruned
